# baseline (speedup 1.0000x reference)
import functools

import jax
import jax.numpy as jnp
from jax import lax
from jax.experimental import pallas as pl
from jax.experimental.pallas import tpu as pltpu
from jax.experimental.pallas import tpu_sc as plsc

_N = 50000
_E = 1600000
_B = 50
_NBUS = 1000
_SN = 1.0
_LAYERS = 5

_NT = 16
_NPAD = 51200
_SL = _NPAD // _NT
_EPT = _E // _NT
_CH = 2000
_NCH = _EPT // _CH
_L = 16
_U = 5
_GI = 128


def _mlp_body(x_ref, w0, b0, w1, b1, w2, b2, w3, b3, w4, b4, th_ref, p_ref):
    x = x_ref[...]
    p_ref[...] = x[:, 0:1] - x[:, 1:2]
    h = x
    for w, b in ((w0, b0), (w1, b1), (w2, b2), (w3, b3)):
        h = jnp.dot(h, w[...], preferred_element_type=jnp.float32) + b[...]
        h = jnp.maximum(h, 0.0)
    th_ref[...] = (jnp.dot(h, w4[...], preferred_element_type=jnp.float32)
                   + b4[...])


def _run_mlp(xpad, wts, bss):
    bn = 12800
    grid = (_NPAD // bn,)
    in_specs = [pl.BlockSpec((bn, 2), lambda i: (i, 0))]
    for w, b in zip(wts, bss):
        in_specs.append(pl.BlockSpec(w.shape, lambda i: (0, 0)))
        in_specs.append(pl.BlockSpec(b.shape, lambda i: (0, 0)))
    args = [xpad]
    for w, b in zip(wts, bss):
        args += [w, b]
    theta, p = pl.pallas_call(
        _mlp_body,
        grid=grid,
        in_specs=in_specs,
        out_specs=[pl.BlockSpec((bn, 1), lambda i: (i, 0))] * 2,
        out_shape=[jax.ShapeDtypeStruct((_NPAD, 1), jnp.float32)] * 2,
    )(*args)
    return theta.reshape(_NPAD), p.reshape(_NPAD)


def _gnn_body(theta0_h, p_h, src_h, dst_h, attr_h, ybus_h, didx_h, ridx_h,
              out_h, parts_h, outa_h,
              theta_v, aggr_v, es0, ed0, ea0, es1, ed1, ea1,
              pslice, invd, maskf, idx_v, sem0, sem1):
    wid = lax.axis_index("s")
    base = wid * _SL

    pltpu.sync_copy(p_h.at[pl.ds(base, _SL)], pslice)
    pltpu.sync_copy(didx_h.at[pl.ds(base, _SL)], idx_v)
    for j in range(_SL // _GI):
        pltpu.async_copy(ybus_h.at[idx_v.at[pl.ds(j * _GI, _GI)]],
                         invd.at[pl.ds(j * _GI, _GI)], sem0)
    for j in range(_SL // _GI):
        pltpu.make_async_copy(ybus_h.at[idx_v.at[pl.ds(j * _GI, _GI)]],
                              invd.at[pl.ds(j * _GI, _GI)], sem0).wait()

    def _prep(g, _):
        sl = pl.ds(g * _L, _L)
        dvec = invd[sl] * _SN
        nz = dvec != 0.0
        invd[sl] = jnp.where(nz, 1.0 / dvec, 0.0)
        maskf[sl] = jnp.where(nz, 1.0, 0.0)
        return 0
    lax.fori_loop(0, _SL // _L, _prep, 0)

    pltpu.sync_copy(ridx_h.at[pl.ds(base, _SL)], idx_v)

    pltpu.sync_copy(theta0_h.at[pl.ds(base, _SL)], pslice)
    pltpu.sync_copy(pslice, out_h.at[pl.ds(base, _SL)])
    pltpu.sync_copy(p_h.at[pl.ds(base, _SL)], pslice)
    plsc.subcore_barrier()

    def _start(c, bs, bd, ba, sem):
        eb = wid * _EPT + c * _CH
        pltpu.async_copy(src_h.at[pl.ds(eb, _CH)], bs, sem)
        pltpu.async_copy(dst_h.at[pl.ds(eb, _CH)], bd, sem)
        pltpu.async_copy(attr_h.at[pl.ds(eb, _CH)], ba, sem)

    def _wait(c, bs, bd, ba, sem):
        eb = wid * _EPT + c * _CH
        pltpu.make_async_copy(src_h.at[pl.ds(eb, _CH)], bs, sem).wait()
        pltpu.make_async_copy(dst_h.at[pl.ds(eb, _CH)], bd, sem).wait()
        pltpu.make_async_copy(attr_h.at[pl.ds(eb, _CH)], ba, sem).wait()

    def _proc(bs, bd, ba):
        def _grp(g2, _):
            for u in range(_U):
                sl = pl.ds((g2 * _U + u) * _L, _L)
                th = plsc.load_gather(theta_v, [bs[sl]])
                plsc.addupdate_scatter(aggr_v, [bd[sl]], th * (ba[sl] * _SN))
            return 0
        lax.fori_loop(0, (_CH // _L) // _U, _grp, 0)

    def _layer(_k, carry):
        pltpu.sync_copy(out_h, theta_v)

        def _zero(g, _):
            for u in range(8):
                aggr_v[pl.ds((g * 8 + u) * _L, _L)] = jnp.zeros((_L,),
                                                                jnp.float32)
            return 0
        lax.fori_loop(0, _NPAD // (_L * 8), _zero, 0)

        _start(0, es0, ed0, ea0, sem0)

        def _c2(c2, _):
            c0 = c2 * 2
            _start(c0 + 1, es1, ed1, ea1, sem1)
            _wait(c0, es0, ed0, ea0, sem0)
            _proc(es0, ed0, ea0)

            @pl.when(c2 < _NCH // 2 - 1)
            def _():
                _start(c0 + 2, es0, ed0, ea0, sem0)
            _wait(c0 + 1, es1, ed1, ea1, sem1)
            _proc(es1, ed1, ea1)
            return 0
        lax.fori_loop(0, _NCH // 2, _c2, 0)

        pltpu.sync_copy(aggr_v, parts_h.at[wid])
        plsc.subcore_barrier()

        for j in range(_NT):
            pltpu.async_copy(parts_h.at[j, pl.ds(base, _SL)],
                             theta_v.at[pl.ds(j * _SL, _SL)], sem0)
        for j in range(_NT):
            pltpu.make_async_copy(parts_h.at[j, pl.ds(base, _SL)],
                                  theta_v.at[pl.ds(j * _SL, _SL)],
                                  sem0).wait()

        def _red(g, _):
            off = g * _L
            v = theta_v[pl.ds(off, _L)]
            for j in range(1, _NT):
                v = v + theta_v[pl.ds(j * _SL + off, _L)]
            aggr_v[pl.ds(off, _L)] = ((pslice[pl.ds(off, _L)] - v)
                                      * invd[pl.ds(off, _L)])
            return 0
        lax.fori_loop(0, _SL // _L, _red, 0)
        pltpu.sync_copy(aggr_v.at[pl.ds(0, _SL)], outa_h.at[pl.ds(base, _SL)])
        plsc.subcore_barrier()

        for j in range(_SL // _GI):
            pltpu.async_copy(outa_h.at[idx_v.at[pl.ds(j * _GI, _GI)]],
                             aggr_v.at[pl.ds(_SL + j * _GI, _GI)], sem0)
        for j in range(_SL // _GI):
            pltpu.make_async_copy(outa_h.at[idx_v.at[pl.ds(j * _GI, _GI)]],
                                  aggr_v.at[pl.ds(_SL + j * _GI, _GI)],
                                  sem0).wait()

        def _fin(g, _):
            off = g * _L
            aggr_v[pl.ds(off, _L)] = ((aggr_v[pl.ds(off, _L)]
                                       - aggr_v[pl.ds(_SL + off, _L)])
                                      * maskf[pl.ds(off, _L)])
            return 0
        lax.fori_loop(0, _SL // _L, _fin, 0)
        pltpu.sync_copy(aggr_v.at[pl.ds(0, _SL)], out_h.at[pl.ds(base, _SL)])
        plsc.subcore_barrier()
        return carry

    lax.fori_loop(0, _LAYERS, _layer, 0)


def _run_gnn(theta0, p, src, dst, attr, ybus_flat, didx, ridx):
    mesh = plsc.VectorSubcoreMesh(core_axis_name="c", subcore_axis_name="s",
                                  num_cores=1)
    f32 = jnp.float32
    kern = pl.kernel(
        _gnn_body,
        out_type=(
            jax.ShapeDtypeStruct((_NPAD,), f32),
            jax.ShapeDtypeStruct((_NT, _NPAD), f32),
            jax.ShapeDtypeStruct((_NPAD,), f32),
        ),
        mesh=mesh,
        compiler_params=pltpu.CompilerParams(needs_layout_passes=False),
        scratch_types=[
            pltpu.VMEM((_NPAD,), f32),
            pltpu.VMEM((_NPAD,), f32),
            pltpu.VMEM((_CH,), jnp.int32),
            pltpu.VMEM((_CH,), jnp.int32),
            pltpu.VMEM((_CH,), f32),
            pltpu.VMEM((_CH,), jnp.int32),
            pltpu.VMEM((_CH,), jnp.int32),
            pltpu.VMEM((_CH,), f32),
            pltpu.VMEM((_SL,), f32),
            pltpu.VMEM((_SL,), f32),
            pltpu.VMEM((_SL,), f32),
            pltpu.VMEM((_SL,), jnp.int32),
            pltpu.SemaphoreType.DMA,
            pltpu.SemaphoreType.DMA,
        ],
    )
    return kern(theta0, p, src, dst, attr, ybus_flat, didx, ridx)


def kernel(x, edge_index_no_diag, edge_attr_no_diag, ybus,
           W0, b0, W1, b1, W2, b2, W3, b3, W4, b4):
    f32 = jnp.float32
    xpad = jnp.pad(x.astype(f32), ((0, _NPAD - _N), (0, 0)))
    wts = [W0.T, W1.T, W2.T, W3.T, W4.T]
    bss = [b.reshape(1, -1) for b in (b0, b1, b2, b3, b4)]
    theta0, p = _run_mlp(xpad, wts, bss)

    src = edge_index_no_diag[0]
    dst = edge_index_no_diag[1]
    attr = edge_attr_no_diag.astype(f32)
    ybus_flat = ybus.reshape(-1).astype(f32)

    i = jnp.arange(_NPAD, dtype=jnp.int32)
    b = i // _NBUS
    r = i - b * _NBUS
    didx = jnp.where(i < _N, b * (_NBUS * _NBUS) + r * _NBUS + r, 0)
    ridx = jnp.where(i < _N, b * _NBUS, 0)

    out, _, _ = _run_gnn(theta0, p, src, dst, attr, ybus_flat, didx, ridx)
    return out[:_N].reshape(_N, 1)


# speedup vs baseline: 1.4239x; 1.4239x over previous
import functools

import jax
import jax.numpy as jnp
from jax import lax
from jax.experimental import pallas as pl
from jax.experimental.pallas import tpu as pltpu
from jax.experimental.pallas import tpu_sc as plsc

_N = 50000
_E = 1600000
_B = 50
_NBUS = 1000
_SN = 1.0
_LAYERS = 5

_NT = 16
_NPAD = 51200
_SL = _NPAD // _NT
_EPT = _E // _NT
_CH = 2000
_NCH = _EPT // _CH
_L = 16
_U = 5
_GI = 128
_DT = 128


def _mlp_body(x_ref, w0, b0, w1, b1, w2, b2, w3, b3, w4, b4, th_ref, p_ref):
    x = x_ref[...]
    p_ref[...] = x[:, 0] - x[:, 1]
    h = x
    for w, b in ((w0, b0), (w1, b1), (w2, b2), (w3, b3)):
        h = jnp.dot(h, w[...], preferred_element_type=jnp.float32) + b[...]
        h = jnp.maximum(h, 0.0)
    h = jnp.dot(h, w4[...], preferred_element_type=jnp.float32) + b4[...]
    th_ref[...] = h[:, 0]


def _run_mlp(x, wts, bss):
    bn = 10240
    grid = (_NPAD // bn,)
    in_specs = [pl.BlockSpec((bn, 2), lambda i: (i, 0))]
    for w, b in zip(wts, bss):
        in_specs.append(pl.BlockSpec(w.shape, lambda i: (0, 0)))
        in_specs.append(pl.BlockSpec(b.shape, lambda i: (0, 0)))
    args = [x]
    for w, b in zip(wts, bss):
        args += [w, b]
    theta, p = pl.pallas_call(
        _mlp_body,
        grid=grid,
        in_specs=in_specs,
        out_specs=[pl.BlockSpec((bn,), lambda i: (i,))] * 2,
        out_shape=[jax.ShapeDtypeStruct((_NPAD,), jnp.float32)] * 2,
    )(*args)
    return theta, p


def _diag_body(yb_ref, d_ref):
    blk = yb_ref[0]
    rr = lax.broadcasted_iota(jnp.int32, (_DT, _DT), 0)
    cc = lax.broadcasted_iota(jnp.int32, (_DT, _DT), 1)
    d = jnp.sum(jnp.where(rr == cc, blk, 0.0), axis=0)
    d_ref[...] = d.reshape(1, 1, 1, _DT)


def _run_diag(ybus):
    nj = 8
    return pl.pallas_call(
        _diag_body,
        grid=(_B, nj),
        in_specs=[pl.BlockSpec((1, _DT, _DT), lambda b, j: (b, j, j))],
        out_specs=pl.BlockSpec((1, 1, 1, _DT), lambda b, j: (b, j, 0, 0)),
        out_shape=jax.ShapeDtypeStruct((_B, nj, 1, _DT), jnp.float32),
    )(ybus)


def _gnn_body(theta0_h, p_h, ei_h, attr_h, diag_h, didx_h, ridx_h,
              out_h, parts_h, outa_h,
              theta_v, aggr_v, es0, ed0, ea0, es1, ed1, ea1,
              pslice, invd, maskf, idx_v, sem0, sem1):
    wid = lax.axis_index("s")
    base = wid * _SL

    pltpu.sync_copy(p_h.at[pl.ds(base, _SL)], pslice)
    pltpu.sync_copy(didx_h.at[pl.ds(base, _SL)], idx_v)
    for j in range(_SL // _GI):
        pltpu.async_copy(diag_h.at[idx_v.at[pl.ds(j * _GI, _GI)]],
                         invd.at[pl.ds(j * _GI, _GI)], sem0)
    for j in range(_SL // _GI):
        pltpu.make_async_copy(diag_h.at[idx_v.at[pl.ds(j * _GI, _GI)]],
                              invd.at[pl.ds(j * _GI, _GI)], sem0).wait()

    def _prep(g, _):
        sl = pl.ds(g * _L, _L)
        dvec = invd[sl] * _SN
        nz = dvec != 0.0
        invd[sl] = jnp.where(nz, 1.0 / dvec, 0.0)
        maskf[sl] = jnp.where(nz, 1.0, 0.0)
        return 0
    lax.fori_loop(0, _SL // _L, _prep, 0)

    pltpu.sync_copy(ridx_h.at[pl.ds(base, _SL)], idx_v)

    pltpu.sync_copy(theta0_h.at[pl.ds(base, _SL)], pslice)
    pltpu.sync_copy(pslice, out_h.at[pl.ds(base, _SL)])
    pltpu.sync_copy(p_h.at[pl.ds(base, _SL)], pslice)
    plsc.subcore_barrier()

    def _start(c, bs, bd, ba, sem):
        eb = wid * _EPT + c * _CH
        pltpu.async_copy(ei_h.at[pl.ds(eb, _CH)], bs, sem)
        pltpu.async_copy(ei_h.at[pl.ds(_E + eb, _CH)], bd, sem)
        pltpu.async_copy(attr_h.at[pl.ds(eb, _CH)], ba, sem)

    def _wait(c, bs, bd, ba, sem):
        eb = wid * _EPT + c * _CH
        pltpu.make_async_copy(ei_h.at[pl.ds(eb, _CH)], bs, sem).wait()
        pltpu.make_async_copy(ei_h.at[pl.ds(_E + eb, _CH)], bd, sem).wait()
        pltpu.make_async_copy(attr_h.at[pl.ds(eb, _CH)], ba, sem).wait()

    def _proc(bs, bd, ba):
        def _grp(g2, _):
            sls = [pl.ds((g2 * _U + u) * _L, _L) for u in range(_U)]
            ss = [bs[sl] for sl in sls]
            dd = [bd[sl] for sl in sls]
            aa = [ba[sl] for sl in sls]
            ths = [plsc.load_gather(theta_v, [s]) for s in ss]
            for u in range(_U):
                plsc.addupdate_scatter(aggr_v, [dd[u]],
                                       ths[u] * (aa[u] * _SN))
            return 0
        lax.fori_loop(0, (_CH // _L) // _U, _grp, 0)

    def _layer(_k, carry):
        pltpu.sync_copy(out_h, theta_v)

        def _zero(g, _):
            for u in range(8):
                aggr_v[pl.ds((g * 8 + u) * _L, _L)] = jnp.zeros((_L,),
                                                                jnp.float32)
            return 0
        lax.fori_loop(0, _NPAD // (_L * 8), _zero, 0)

        _start(0, es0, ed0, ea0, sem0)

        def _c2(c2, _):
            c0 = c2 * 2
            _start(c0 + 1, es1, ed1, ea1, sem1)
            _wait(c0, es0, ed0, ea0, sem0)
            _proc(es0, ed0, ea0)

            @pl.when(c2 < _NCH // 2 - 1)
            def _():
                _start(c0 + 2, es0, ed0, ea0, sem0)
            _wait(c0 + 1, es1, ed1, ea1, sem1)
            _proc(es1, ed1, ea1)
            return 0
        lax.fori_loop(0, _NCH // 2, _c2, 0)

        pltpu.sync_copy(aggr_v, parts_h.at[wid])
        plsc.subcore_barrier()

        for j in range(_NT):
            pltpu.async_copy(parts_h.at[j, pl.ds(base, _SL)],
                             theta_v.at[pl.ds(j * _SL, _SL)], sem0)
        for j in range(_NT):
            pltpu.make_async_copy(parts_h.at[j, pl.ds(base, _SL)],
                                  theta_v.at[pl.ds(j * _SL, _SL)],
                                  sem0).wait()

        def _red(g, _):
            off = g * _L
            v = theta_v[pl.ds(off, _L)]
            for j in range(1, _NT):
                v = v + theta_v[pl.ds(j * _SL + off, _L)]
            aggr_v[pl.ds(off, _L)] = ((pslice[pl.ds(off, _L)] - v)
                                      * invd[pl.ds(off, _L)])
            return 0
        lax.fori_loop(0, _SL // _L, _red, 0)
        pltpu.sync_copy(aggr_v.at[pl.ds(0, _SL)], outa_h.at[pl.ds(base, _SL)])
        plsc.subcore_barrier()

        for j in range(_SL // _GI):
            pltpu.async_copy(outa_h.at[idx_v.at[pl.ds(j * _GI, _GI)]],
                             aggr_v.at[pl.ds(_SL + j * _GI, _GI)], sem0)
        for j in range(_SL // _GI):
            pltpu.make_async_copy(outa_h.at[idx_v.at[pl.ds(j * _GI, _GI)]],
                                  aggr_v.at[pl.ds(_SL + j * _GI, _GI)],
                                  sem0).wait()

        def _fin(g, _):
            off = g * _L
            aggr_v[pl.ds(off, _L)] = ((aggr_v[pl.ds(off, _L)]
                                       - aggr_v[pl.ds(_SL + off, _L)])
                                      * maskf[pl.ds(off, _L)])
            return 0
        lax.fori_loop(0, _SL // _L, _fin, 0)
        pltpu.sync_copy(aggr_v.at[pl.ds(0, _SL)], out_h.at[pl.ds(base, _SL)])
        plsc.subcore_barrier()
        return carry

    lax.fori_loop(0, _LAYERS, _layer, 0)


def _run_gnn(theta0, p, edge_index, attr, diagflat, didx, ridx):
    mesh = plsc.VectorSubcoreMesh(core_axis_name="c", subcore_axis_name="s",
                                  num_cores=1)
    f32 = jnp.float32
    kern = pl.kernel(
        _gnn_body,
        out_type=(
            jax.ShapeDtypeStruct((_NPAD,), f32),
            jax.ShapeDtypeStruct((_NT, _NPAD), f32),
            jax.ShapeDtypeStruct((_NPAD,), f32),
        ),
        mesh=mesh,
        compiler_params=pltpu.CompilerParams(needs_layout_passes=False),
        scratch_types=[
            pltpu.VMEM((_NPAD,), f32),
            pltpu.VMEM((_NPAD,), f32),
            pltpu.VMEM((_CH,), jnp.int32),
            pltpu.VMEM((_CH,), jnp.int32),
            pltpu.VMEM((_CH,), f32),
            pltpu.VMEM((_CH,), jnp.int32),
            pltpu.VMEM((_CH,), jnp.int32),
            pltpu.VMEM((_CH,), f32),
            pltpu.VMEM((_SL,), f32),
            pltpu.VMEM((_SL,), f32),
            pltpu.VMEM((_SL,), f32),
            pltpu.VMEM((_SL,), jnp.int32),
            pltpu.SemaphoreType.DMA,
            pltpu.SemaphoreType.DMA,
        ],
    )
    return kern(theta0, p, edge_index, attr, diagflat, didx, ridx)


def kernel(x, edge_index_no_diag, edge_attr_no_diag, ybus,
           W0, b0, W1, b1, W2, b2, W3, b3, W4, b4):
    f32 = jnp.float32
    wts = [W0.T, W1.T, W2.T, W3.T, W4.T]
    bss = [b.reshape(1, -1) for b in (b0, b1, b2, b3, b4)]
    theta0, p = _run_mlp(x.astype(f32), wts, bss)

    attr = edge_attr_no_diag.astype(f32)
    diagflat = _run_diag(ybus.astype(f32)).reshape(_B * 8 * _DT)

    i = jnp.arange(_NPAD, dtype=jnp.int32)
    b = i // _NBUS
    r = i - b * _NBUS
    didx = jnp.where(i < _N, b * (8 * _DT) + r, 0)
    ridx = jnp.where(i < _N, b * _NBUS, 0)

    out, _, _ = _run_gnn(theta0, p, edge_index_no_diag.reshape(2 * _E),
                         attr, diagflat, didx, ridx)
    return out[:_N].reshape(_N, 1)
